# Initial kernel scaffold; baseline (speedup 1.0000x reference)
import jax
import jax.numpy as jnp
from jax.experimental import pallas as pl


def kernel(x, params, edge_index, batch):
    raise NotImplementedError("write your pallas kernel here")


# speedup vs baseline: 8.5723x; 8.5723x over previous
import functools

import jax
import jax.numpy as jnp
from jax import lax
from jax.experimental import pallas as pl
from jax.experimental.pallas import tpu as pltpu
from jax.experimental.pallas import tpu_sc as plsc

_N = 10000
_E2 = 330000
_G = 64
_NC = 2
_NS = 16
_NW = _NC * _NS
_B = 64
_EPW = 10368
_EP = _EPW * _NW
_RPT = 624


@functools.lru_cache(maxsize=None)
def _edge_call(dout):
    J = dout // 16
    W = dout + 16
    mesh = plsc.VectorSubcoreMesh(core_axis_name="c", subcore_axis_name="s")

    @functools.partial(
        pl.kernel,
        out_type=jax.ShapeDtypeStruct((_NC, _N, W), jnp.float32),
        mesh=mesh,
        compiler_params=pltpu.CompilerParams(needs_layout_passes=False,
                                             use_tc_tiling_on_sc=False),
        scratch_types=[
            pltpu.VMEM_SHARED((_N, W), jnp.float32),
            pltpu.VMEM((1, _B), jnp.int32),
            pltpu.VMEM((1, _B), jnp.int32),
            pltpu.VMEM((_B, dout), jnp.float32),
            pltpu.VMEM((_B, W), jnp.float32),
            pltpu.VMEM((_B, W), jnp.float32),
            pltpu.VMEM((dout,), jnp.float32),
            pltpu.SemaphoreType.DMA,
            pltpu.SemaphoreType.DMA,
        ],
    )
    def k(xl_hbm, xrm_hbm, src_hbm, dst_hbm, att_hbm, out_hbm,
          acc_sp, sidx, didx, xlr, xrr, outr, attv, sem1, sem2):
        c = lax.axis_index("c")
        s = lax.axis_index("s")
        wid = c * _NS + s
        zv = jnp.zeros((16,), jnp.float32)

        def zrow(r, carry):
            for jw in range(W // 16):
                outr[r, pl.ds(jw * 16, 16)] = zv
            return carry

        lax.fori_loop(0, _B, zrow, 0)
        r0 = pl.multiple_of(s * _RPT, 8)
        nfull = _RPT // _B
        rem = _RPT - nfull * _B
        for kk in range(nfull):
            pltpu.sync_copy(outr, acc_sp.at[pl.ds(r0 + kk * _B, _B)])
        if rem:
            pltpu.sync_copy(outr.at[pl.ds(0, rem)],
                            acc_sp.at[pl.ds(r0 + nfull * _B, rem)])
        tail = _N - _NS * _RPT

        @pl.when(s == _NS - 1)
        def _():
            pltpu.sync_copy(outr.at[pl.ds(0, tail)],
                            acc_sp.at[pl.ds(_NS * _RPT, tail)])

        plsc.subcore_barrier()

        pltpu.sync_copy(att_hbm, attv)
        lanes = lax.iota(jnp.int32, 16)

        def gbody(g, carry):
            base = wid * _EPW + g * _B
            pltpu.sync_copy(src_hbm.at[pl.ds(base, _B)], sidx.at[0])
            pltpu.sync_copy(dst_hbm.at[pl.ds(base, _B)], didx.at[0])
            pltpu.async_copy(xl_hbm.at[sidx.at[0]], xlr, sem1).wait()
            pltpu.async_copy(xrm_hbm.at[didx.at[0]], xrr, sem2).wait()
            att = [attv[pl.ds(j * 16, 16)] for j in range(J)]

            def ebody(e, carry2):
                avs = []
                accv = jnp.zeros((16,), jnp.float32)
                for j in range(J):
                    a = xlr[e, pl.ds(j * 16, 16)]
                    b = xrr[e, pl.ds(j * 16, 16)]
                    avs.append(a)
                    v = a + b
                    u = jnp.maximum(v, 0.2 * v)
                    accv = accv + u * att[j]
                logit = jnp.sum(accv)
                mv = xrr[e, pl.ds(dout, 16)]
                d = jnp.clip(jnp.full((16,), logit, jnp.float32) - mv,
                             -60.0, 60.0)
                ex = jnp.exp(d)
                eidv = jnp.full((16,), base + e, jnp.int32)
                ex = jnp.where(eidv < _E2, ex, 0.0)
                for j in range(J):
                    outr[e, pl.ds(j * 16, 16)] = avs[j] * ex
                outr[e, pl.ds(dout, 16)] = jnp.where(lanes == 0, ex, 0.0)
                return carry2

            lax.fori_loop(0, _B, ebody, 0)
            pltpu.sync_copy(outr, acc_sp.at[didx.at[0]], add=True)
            return carry

        lax.fori_loop(0, _EPW // _B, gbody, 0)
        plsc.subcore_barrier()
        for kk in range(nfull):
            pltpu.sync_copy(acc_sp.at[pl.ds(r0 + kk * _B, _B)],
                            out_hbm.at[c, pl.ds(r0 + kk * _B, _B)])
        if rem:
            pltpu.sync_copy(acc_sp.at[pl.ds(r0 + nfull * _B, rem)],
                            out_hbm.at[c, pl.ds(r0 + nfull * _B, rem)])

        @pl.when(s == _NS - 1)
        def _():
            pltpu.sync_copy(acc_sp.at[pl.ds(_NS * _RPT, tail)],
                            out_hbm.at[c, pl.ds(_NS * _RPT, tail)])

    return k


@functools.lru_cache(maxsize=None)
def _proj_call(din, dout):
    W = dout + 16

    def body(h_ref, wl_ref, bl_ref, wr_ref, br_ref, att_ref,
             xl_ref, xrm_ref):
        h = h_ref[...]
        xl = jnp.dot(h, wl_ref[...],
                     preferred_element_type=jnp.float32) + bl_ref[...]
        xr = jnp.dot(h, wr_ref[...],
                     preferred_element_type=jnp.float32) + br_ref[...]
        v = xl + xr
        u = jnp.maximum(v, 0.2 * v)
        m = jnp.sum(u * att_ref[...], axis=1, keepdims=True)
        xl_ref[...] = xl
        xrm_ref[...] = jnp.concatenate(
            [xr, jnp.broadcast_to(m, (_N, 16))], axis=1)

    return pl.pallas_call(
        body,
        out_shape=(jax.ShapeDtypeStruct((_N, dout), jnp.float32),
                   jax.ShapeDtypeStruct((_N, W), jnp.float32)),
    )


@functools.lru_cache(maxsize=None)
def _post_call(dout):
    W = dout + 16

    def body(ad_ref, b_ref, g_ref, bb_ref, batch_ref, h_ref, p_ref):
        sacc = ad_ref[0] + ad_ref[1]
        den = sacc[:, dout:dout + 1]
        out = sacc[:, :dout] / den + b_ref[...]
        h0 = jnp.maximum(out, 0.0)
        mu = jnp.mean(h0, axis=0, keepdims=True)
        var = jnp.mean((h0 - mu) ** 2, axis=0, keepdims=True)
        h = g_ref[...] * (h0 - mu) * lax.rsqrt(var + 1e-5) + bb_ref[...]
        h_ref[...] = h
        onehot = (batch_ref[...] == lax.broadcasted_iota(
            jnp.int32, (_N, _G), 1)).astype(jnp.float32)
        p_ref[...] = lax.dot_general(
            onehot, h, (((0,), (0,)), ((), ())),
            preferred_element_type=jnp.float32)

    return pl.pallas_call(
        body,
        out_shape=(jax.ShapeDtypeStruct((_N, dout), jnp.float32),
                   jax.ShapeDtypeStruct((_G, dout), jnp.float32)),
    )


def _head_body(p1_ref, p2_ref, p3_ref, w1_ref, b1_ref, g_ref, bb_ref,
               w2_ref, b2_ref, sig_ref, lsm_ref):
    h = jnp.concatenate(
        [p1_ref[...], p2_ref[...], p3_ref[...], p3_ref[...]], axis=1)
    z = jnp.dot(h, w1_ref[...],
                preferred_element_type=jnp.float32) + b1_ref[...]
    z = jnp.maximum(z, 0.0)
    mu = jnp.mean(z, axis=0, keepdims=True)
    var = jnp.mean((z - mu) ** 2, axis=0, keepdims=True)
    z = g_ref[...] * (z - mu) * lax.rsqrt(var + 1e-5) + bb_ref[...]
    o = jnp.dot(z, w2_ref[...],
                preferred_element_type=jnp.float32) + b2_ref[...]
    sig_ref[...] = 1.0 / (1.0 + jnp.exp(-o))
    om = jnp.max(o, axis=1, keepdims=True)
    lse = om + jnp.log(jnp.sum(jnp.exp(o - om), axis=1, keepdims=True))
    lsm_ref[...] = o - lse


_head_call = pl.pallas_call(
    _head_body,
    out_shape=(jax.ShapeDtypeStruct((_G, 10), jnp.float32),
               jax.ShapeDtypeStruct((_G, 10), jnp.float32)),
)


def kernel(x, params, edge_index, batch):
    loop = jnp.arange(_N, dtype=edge_index.dtype)
    pad = jnp.arange(_EP - _E2, dtype=edge_index.dtype) % _N
    src = jnp.concatenate([edge_index[0], loop, pad])
    dst = jnp.concatenate([edge_index[1], loop, pad])
    batch2 = batch.reshape(_N, 1)

    h = x
    pooled = []
    for i, (din, dout) in enumerate(((128, 128), (128, 64), (64, 32)),
                                    start=1):
        att = params['gat%d_att' % i]
        xl, xrm = _proj_call(din, dout)(
            h, params['gat%d_Wl' % i].T,
            params['gat%d_bl' % i].reshape(1, -1),
            params['gat%d_Wr' % i].T,
            params['gat%d_br' % i].reshape(1, -1),
            att.reshape(1, -1))
        accden = _edge_call(dout)(xl, xrm, src, dst, att)
        h, p = _post_call(dout)(
            accden, params['gat%d_b' % i].reshape(1, -1),
            params['bn%d_g' % i].reshape(1, -1),
            params['bn%d_b' % i].reshape(1, -1), batch2)
        pooled.append(p)

    return _head_call(
        pooled[0], pooled[1], pooled[2],
        params['lin1_W'].T, params['lin1_b'].reshape(1, -1),
        params['bn5_g'].reshape(1, -1), params['bn5_b'].reshape(1, -1),
        params['lin2_W'].T, params['lin2_b'].reshape(1, -1))


# speedup vs baseline: 16.3265x; 1.9046x over previous
import functools

import jax
import jax.numpy as jnp
from jax import lax
from jax.experimental import pallas as pl
from jax.experimental.pallas import tpu as pltpu
from jax.experimental.pallas import tpu_sc as plsc

_N = 10000
_E2 = 330000
_G = 64
_NC = 2
_NS = 16
_NW = _NC * _NS
_EPW = 10496
_EP = _EPW * _NW
_RPT = 624


@functools.lru_cache(maxsize=None)
def _edge_call(dout):
    J = dout // 16
    W = dout + 16
    B = 32 if dout == 128 else 128
    NB = _EPW // B
    NT = NB // 2
    mesh = plsc.VectorSubcoreMesh(core_axis_name="c", subcore_axis_name="s")

    @functools.partial(
        pl.kernel,
        out_type=jax.ShapeDtypeStruct((_NC, _N, W), jnp.float32),
        mesh=mesh,
        compiler_params=pltpu.CompilerParams(needs_layout_passes=False,
                                             use_tc_tiling_on_sc=False),
        scratch_types=[
            pltpu.VMEM_SHARED((_N, W), jnp.float32),
            pltpu.VMEM((NB, B), jnp.int32),
            pltpu.VMEM((NB, B), jnp.int32),
            pltpu.VMEM((2, B, W), jnp.float32),
            pltpu.VMEM((2, B, W), jnp.float32),
            pltpu.VMEM((dout,), jnp.float32),
            pltpu.SemaphoreType.DMA,
            pltpu.SemaphoreType.DMA,
            pltpu.SemaphoreType.DMA,
            pltpu.SemaphoreType.DMA,
        ],
    )
    def k(xl_hbm, xrm_hbm, src_hbm, dst_hbm, att_hbm, out_hbm,
          acc_sp, sidx, didx, xrr, outr, attv, gs0, gs1, ss0, ss1):
        c = lax.axis_index("c")
        s = lax.axis_index("s")
        wid = c * _NS + s
        zv = jnp.zeros((16,), jnp.float32)
        gsem = (gs0, gs1)
        scsem = (ss0, ss1)

        def zrow(r, carry):
            for jw in range(W // 16):
                outr[0, r, pl.ds(jw * 16, 16)] = zv
            return carry

        lax.fori_loop(0, B, zrow, 0)
        r0 = pl.multiple_of(s * _RPT, 8)
        zstep = 48 if B >= 48 else 16
        nz = _RPT // zstep
        assert nz * zstep == _RPT
        tail = _N - _NS * _RPT

        def zcopy(i, carry):
            pltpu.sync_copy(outr.at[0, pl.ds(0, zstep)],
                            acc_sp.at[pl.ds(pl.multiple_of(r0 + i * zstep, 8),
                                            zstep)])
            return carry

        lax.fori_loop(0, nz, zcopy, 0)

        @pl.when(s == _NS - 1)
        def _():
            pltpu.sync_copy(outr.at[0, pl.ds(0, tail)],
                            acc_sp.at[pl.ds(_NS * _RPT, tail)])

        plsc.subcore_barrier()

        pltpu.sync_copy(att_hbm, attv)
        pltpu.sync_copy(src_hbm.at[wid], sidx)
        pltpu.sync_copy(dst_hbm.at[wid], didx)
        att = [attv[pl.ds(j * 16, 16)] for j in range(J)]

        def start_gather(g, p):
            pltpu.async_copy(xl_hbm.at[sidx.at[g]], outr.at[p], gsem[p])
            pltpu.async_copy(xrm_hbm.at[didx.at[g]], xrr.at[p], gsem[p])

        def wait_gather(g, p):
            pltpu.make_async_copy(xl_hbm.at[sidx.at[g]], outr.at[p],
                                  gsem[p]).wait()
            pltpu.make_async_copy(xrm_hbm.at[didx.at[g]], xrr.at[p],
                                  gsem[p]).wait()

        def wait_scatter(g, p):
            pltpu.make_async_copy(outr.at[p], acc_sp.at[didx.at[g]],
                                  scsem[p]).wait()

        def compute(g, p):
            base = wid * _EPW + g * B

            def ebody(e, carry):
                avs = []
                accv = jnp.zeros((16,), jnp.float32)
                for j in range(J):
                    a = outr[p, e, pl.ds(j * 16, 16)]
                    b = xrr[p, e, pl.ds(j * 16, 16)]
                    avs.append(a)
                    v = a + b
                    u = jnp.maximum(v, 0.2 * v)
                    accv = accv + u * att[j]
                logit = jnp.sum(accv)
                mv = xrr[p, e, pl.ds(dout, 16)]
                d = jnp.clip(jnp.full((16,), logit, jnp.float32) - mv,
                             -60.0, 60.0)
                ex = jnp.exp(d)
                eidv = jnp.full((16,), base + e, jnp.int32)
                ex = jnp.where(eidv < _E2, ex, 0.0)
                for j in range(J):
                    outr[p, e, pl.ds(j * 16, 16)] = avs[j] * ex
                outr[p, e, pl.ds(dout, 16)] = ex
                return carry

            lax.fori_loop(0, B, ebody, 0)

        start_gather(0, 0)

        def tbody(t, carry):
            for b in (0, 1):
                p = b
                g = 2 * t + b
                if b == 0:
                    @pl.when(t > 0)
                    def _():
                        wait_scatter(g - 1, 1 - p)

                    start_gather(g + 1, 1 - p)
                else:
                    wait_scatter(g - 1, 1 - p)

                    @pl.when(t < NT - 1)
                    def _():
                        start_gather(g + 1, 1 - p)
                wait_gather(g, p)
                compute(g, p)
                pltpu.async_copy(outr.at[p], acc_sp.at[didx.at[g]],
                                 scsem[p], add=True)
            return carry

        lax.fori_loop(0, NT, tbody, 0)
        wait_scatter(NB - 1, 1)
        plsc.subcore_barrier()
        for kk in range(_RPT // 208):
            pltpu.sync_copy(
                acc_sp.at[pl.ds(pl.multiple_of(r0 + kk * 208, 8), 208)],
                out_hbm.at[c, pl.ds(pl.multiple_of(r0 + kk * 208, 8), 208)])

        @pl.when(s == _NS - 1)
        def _():
            pltpu.sync_copy(acc_sp.at[pl.ds(_NS * _RPT, tail)],
                            out_hbm.at[c, pl.ds(_NS * _RPT, tail)])

    return k


@functools.lru_cache(maxsize=None)
def _proj_call(din, dout):
    W = dout + 16

    def body(h_ref, wl_ref, bl_ref, wr_ref, br_ref, att_ref,
             xl_ref, xrm_ref):
        h = h_ref[...]
        xl = jnp.dot(h, wl_ref[...],
                     preferred_element_type=jnp.float32) + bl_ref[...]
        xr = jnp.dot(h, wr_ref[...],
                     preferred_element_type=jnp.float32) + br_ref[...]
        v = xl + xr
        u = jnp.maximum(v, 0.2 * v)
        m = jnp.sum(u * att_ref[...], axis=1, keepdims=True)
        xl_ref[...] = jnp.concatenate(
            [xl, jnp.ones((_N, 16), jnp.float32)], axis=1)
        xrm_ref[...] = jnp.concatenate(
            [xr, jnp.broadcast_to(m, (_N, 16))], axis=1)

    return pl.pallas_call(
        body,
        out_shape=(jax.ShapeDtypeStruct((_N, W), jnp.float32),
                   jax.ShapeDtypeStruct((_N, W), jnp.float32)),
    )


@functools.lru_cache(maxsize=None)
def _post_call(dout):
    W = dout + 16

    def body(ad_ref, b_ref, g_ref, bb_ref, batch_ref, h_ref, p_ref):
        sacc = ad_ref[0] + ad_ref[1]
        den = sacc[:, dout:dout + 1]
        out = sacc[:, :dout] / den + b_ref[...]
        h0 = jnp.maximum(out, 0.0)
        mu = jnp.mean(h0, axis=0, keepdims=True)
        var = jnp.mean((h0 - mu) ** 2, axis=0, keepdims=True)
        h = g_ref[...] * (h0 - mu) * lax.rsqrt(var + 1e-5) + bb_ref[...]
        h_ref[...] = h
        onehot = (batch_ref[...] == lax.broadcasted_iota(
            jnp.int32, (_N, _G), 1)).astype(jnp.float32)
        p_ref[...] = lax.dot_general(
            onehot, h, (((0,), (0,)), ((), ())),
            preferred_element_type=jnp.float32)

    return pl.pallas_call(
        body,
        out_shape=(jax.ShapeDtypeStruct((_N, dout), jnp.float32),
                   jax.ShapeDtypeStruct((_G, dout), jnp.float32)),
    )


def _head_body(p1_ref, p2_ref, p3_ref, w1_ref, b1_ref, g_ref, bb_ref,
               w2_ref, b2_ref, sig_ref, lsm_ref):
    h = jnp.concatenate(
        [p1_ref[...], p2_ref[...], p3_ref[...], p3_ref[...]], axis=1)
    z = jnp.dot(h, w1_ref[...],
                preferred_element_type=jnp.float32) + b1_ref[...]
    z = jnp.maximum(z, 0.0)
    mu = jnp.mean(z, axis=0, keepdims=True)
    var = jnp.mean((z - mu) ** 2, axis=0, keepdims=True)
    z = g_ref[...] * (z - mu) * lax.rsqrt(var + 1e-5) + bb_ref[...]
    o = jnp.dot(z, w2_ref[...],
                preferred_element_type=jnp.float32) + b2_ref[...]
    sig_ref[...] = 1.0 / (1.0 + jnp.exp(-o))
    om = jnp.max(o, axis=1, keepdims=True)
    lse = om + jnp.log(jnp.sum(jnp.exp(o - om), axis=1, keepdims=True))
    lsm_ref[...] = o - lse


_head_call = pl.pallas_call(
    _head_body,
    out_shape=(jax.ShapeDtypeStruct((_G, 10), jnp.float32),
               jax.ShapeDtypeStruct((_G, 10), jnp.float32)),
)


def kernel(x, params, edge_index, batch):
    loop = jnp.arange(_N, dtype=edge_index.dtype)
    pad = jnp.arange(_EP - _E2, dtype=edge_index.dtype) % _N
    src = jnp.concatenate([edge_index[0], loop, pad])
    dst = jnp.concatenate([edge_index[1], loop, pad])
    batch2 = batch.reshape(_N, 1)

    h = x
    pooled = []
    for i, (din, dout) in enumerate(((128, 128), (128, 64), (64, 32)),
                                    start=1):
        att = params['gat%d_att' % i]
        xl, xrm = _proj_call(din, dout)(
            h, params['gat%d_Wl' % i].T,
            params['gat%d_bl' % i].reshape(1, -1),
            params['gat%d_Wr' % i].T,
            params['gat%d_br' % i].reshape(1, -1),
            att.reshape(1, -1))
        bsz = 32 if dout == 128 else 128
        accden = _edge_call(dout)(
            xl, xrm, src.reshape(_NW, _EPW // bsz, bsz),
            dst.reshape(_NW, _EPW // bsz, bsz), att)
        h, p = _post_call(dout)(
            accden, params['gat%d_b' % i].reshape(1, -1),
            params['bn%d_g' % i].reshape(1, -1),
            params['bn%d_b' % i].reshape(1, -1), batch2)
        pooled.append(p)

    return _head_call(
        pooled[0], pooled[1], pooled[2],
        params['lin1_W'].T, params['lin1_b'].reshape(1, -1),
        params['bn5_g'].reshape(1, -1), params['bn5_b'].reshape(1, -1),
        params['lin2_W'].T, params['lin2_b'].reshape(1, -1))
